# baseline (speedup 1.0000x reference)
import functools

import numpy as np

import jax
import jax.numpy as jnp
from jax import lax
from jax.experimental import pallas as pl
from jax.experimental.pallas import tpu as pltpu
from jax.experimental.pallas import tpu_sc as plsc

N_DISEASES = 2048
K = 50
D = 512
_NC, _NS = 2, 16
_NW = _NC * _NS
_NEG = np.float32(-np.inf)


def _sim_body(xr_ref, xc_ref, o_ref):
    o_ref[...] = lax.dot_general(
        xr_ref[...], xc_ref[...], (((1,), (1,)), ((), ())),
        preferred_element_type=jnp.float32)


def _similarity(x):
    n = x.shape[0]
    blk = 512
    return pl.pallas_call(
        _sim_body,
        grid=(n // blk, n // blk),
        in_specs=[
            pl.BlockSpec((blk, D), lambda i, j: (i, 0)),
            pl.BlockSpec((blk, D), lambda i, j: (j, 0)),
        ],
        out_specs=pl.BlockSpec((blk, blk), lambda i, j: (i, j)),
        out_shape=jax.ShapeDtypeStruct((n, n), jnp.float32),
    )(x, x)


def _cm_body(s_ref, cm_ref):
    blk, n = s_ref.shape
    cm_ref[...] = jnp.max(s_ref[...].reshape(blk, n // 16, 16), axis=-1)


def _chunk_max(s_mat):
    n = s_mat.shape[0]
    blk = 128
    return pl.pallas_call(
        _cm_body,
        grid=(n // blk,),
        in_specs=[pl.BlockSpec((blk, n), lambda i: (i, 0))],
        out_specs=pl.BlockSpec((blk, n // 16), lambda i: (i, 0)),
        out_shape=jax.ShapeDtypeStruct((n, n // 16), jnp.float32),
    )(s_mat)


def _s16(x):
    return plsc.sort_key_val(x, x, descending=True)[0]


def _rev(x):
    return jnp.flip(x, 0)


def _merge2(a, b):
    rb = _rev(b)
    return _s16(jnp.maximum(a, rb)), _s16(jnp.minimum(a, rb))


def _bm32(x0, x1):
    return _s16(jnp.maximum(x0, x1)), _s16(jnp.minimum(x0, x1))


def _merge4(a, b):
    rb0, rb1 = _rev(b[1]), _rev(b[0])
    h0, h1 = jnp.maximum(a[0], rb0), jnp.maximum(a[1], rb1)
    l0, l1 = jnp.minimum(a[0], rb0), jnp.minimum(a[1], rb1)
    return _bm32(h0, h1) + _bm32(l0, l1)


def _sort64(r):
    s = [_s16(x) for x in r]
    return _merge4(_merge2(s[0], s[1]), _merge2(s[2], s[3]))


def _merge8_top4(a, b):
    rb = [_rev(b[3]), _rev(b[2]), _rev(b[1]), _rev(b[0])]
    h = [jnp.maximum(a[j], rb[j]) for j in range(4)]
    p0, p1 = jnp.maximum(h[0], h[2]), jnp.maximum(h[1], h[3])
    q0, q1 = jnp.minimum(h[0], h[2]), jnp.minimum(h[1], h[3])
    return _bm32(p0, p1) + _bm32(q0, q1)


def _top64(regs):
    t = _sort64(regs[0:4])
    for base in range(4, len(regs), 4):
        t = _merge8_top4(t, _sort64(regs[base:base + 4]))
    return t


def _sc_select(s_mat, cm_mat):
    m, n = s_mat.shape
    rpt = m // _NW
    nch = n // 16
    ncm = nch // 16
    mesh = plsc.VectorSubcoreMesh(core_axis_name="c", subcore_axis_name="s")

    STR = 32
    BUF = 16 * STR

    @functools.partial(
        pl.kernel, mesh=mesh,
        out_type=jax.ShapeDtypeStruct((m, 16), jnp.float32),
        compiler_params=pltpu.CompilerParams(needs_layout_passes=False),
        scratch_types=[pltpu.VMEM((2, n), jnp.float32),
                       pltpu.VMEM((2, nch), jnp.float32),
                       pltpu.VMEM((BUF,), jnp.float32),
                       pltpu.VMEM((192,), jnp.float32),
                       pltpu.VMEM((rpt, 16), jnp.float32),
                       pltpu.SemaphoreType.DMA,
                       pltpu.SemaphoreType.DMA,
                       pltpu.SemaphoreType.DMA,
                       pltpu.SemaphoreType.DMA])
    def _sel(s_hbm, cm_hbm, res_hbm, rowbuf, cmbuf, buf, lst, resv,
             sem0, sem1, sem2, sem3):
        wid = lax.axis_index("s") * _NC + lax.axis_index("c")
        row0 = wid * rpt
        iota = lax.iota(jnp.int32, 16)
        sems = (sem0, sem1)
        csems = (sem2, sem3)
        neg16 = jnp.full((16,), _NEG, jnp.float32)

        def bfly(v, op):
            for s in (8, 4, 2, 1):
                v = op(v, v[iota ^ s])
            return v

        def ins4(L, v):
            out = []
            for j in range(4):
                out.append(jnp.maximum(L[j], v))
                v = jnp.minimum(L[j], v)
            return out

        def top64_of_list(off):
            regs = []
            for j in range(12):
                v = lst[pl.ds(j * 16, 16)]
                regs.append(jnp.where(iota + j * 16 < off, v, _NEG))
            return _top64(regs)

        def lane(vec, k):
            return jnp.max(jnp.where(iota == k, vec, _NEG))

        def fallback_t(b):
            def chunk2(g, carry):
                theta, off = carry
                for u in range(2):
                    v = rowbuf[b, pl.ds((2 * g + u) * 16, 16)]
                    msk = v > theta
                    sk = plsc.sort_key_val(v, v, mask=msk,
                                           descending=True)[0]
                    lst[pl.ds(off, 16)] = sk
                    off = off + jnp.sum(msk.astype(jnp.int32))

                def compact(th2, of2):
                    top = top64_of_list(of2)
                    for j in range(4):
                        lst[pl.ds(j * 16, 16)] = top[j]
                    return lane(top[3], 2), np.int32(64)

                return lax.cond(off > 128, compact,
                                lambda th2, of2: (th2, of2), theta, off)

            theta, off = lax.fori_loop(0, nch // 2, chunk2,
                                       (_NEG, np.int32(0)))
            top = top64_of_list(off)
            return jnp.maximum(lane(top[3], 2), theta)

        def process(b, row_i):
            top4 = [neg16] * 4
            for g in range(ncm):
                top4 = ins4(top4, cmbuf[b, pl.ds(g * 16, 16)])
            th = bfly(top4[3], jnp.minimum)
            m1s = bfly(top4[0], jnp.maximum)

            for j in range(BUF // 16):
                buf[pl.ds(j * 16, 16)] = neg16

            def step_b(i, cnt):
                v = rowbuf[b, pl.ds(i * 16, 16)]
                msk = (v > th) & (cnt < STR)
                idx = cnt * 16 + iota
                plsc.store_scatter(buf, [idx], v, mask=msk)
                return cnt + msk.astype(jnp.int32)

            cnt = lax.fori_loop(0, nch, step_b,
                                jnp.zeros((16,), jnp.int32), unroll=4)
            c_tot = bfly(cnt, jnp.add)[0]
            ov = bfly(cnt, jnp.maximum)[0]

            def sel_from(nreg):
                def f():
                    regs = [buf[pl.ds(j * 16, 16)] for j in range(nreg)]
                    return lane(_top64(regs)[3], 2)
                return f

            def t_main():
                return lax.cond(
                    c_tot > 50,
                    lambda: lax.cond(ov <= 8, sel_from(8),
                                     lambda: lax.cond(ov <= 16, sel_from(16),
                                                      sel_from(STR))),
                    lambda: th[0])

            t = lax.cond(ov >= STR, lambda: fallback_t(b), t_main)
            res = jnp.where(iota == 0, m1s, jnp.where(iota == 1, t, 0.0))
            resv[row_i, :] = res

        def start(r, b):
            pltpu.async_copy(s_hbm.at[row0 + r], rowbuf.at[b], sems[b])
            pltpu.async_copy(cm_hbm.at[row0 + r], cmbuf.at[b], csems[b])

        def wait(b):
            pltpu.make_async_copy(s_hbm.at[0], rowbuf.at[b], sems[b]).wait()
            pltpu.make_async_copy(cm_hbm.at[0], cmbuf.at[b], csems[b]).wait()

        start(0, 0)

        def pair(g, _):
            r = 2 * g
            start(r + 1, 1)
            wait(0)
            process(0, r)

            @pl.when(r + 2 < rpt)
            def _():
                start(r + 2, 0)

            wait(1)
            process(1, r + 1)
            return 0

        lax.fori_loop(0, rpt // 2, pair, 0)
        pltpu.sync_copy(resv, res_hbm.at[pl.ds(row0, rpt)])

    return _sel(s_mat, cm_mat)


def _fd_body(s_ref, t_ref, m1_ref, x_ref, fd_ref, cs_ref):
    i, j = pl.program_id(0), pl.program_id(1)
    s = s_ref[...]
    t = t_ref[...].reshape(-1, 1)
    m1 = m1_ref[...].reshape(-1, 1)
    msk = ((s >= t) & (s != m1)).astype(jnp.float32)
    part = lax.dot_general(msk, x_ref[...], (((1,), (0,)), ((), ())),
                           preferred_element_type=jnp.float32) * (1.0 / K)

    @pl.when(j == 0)
    def _():
        fd_ref[...] = jnp.zeros_like(fd_ref)

    fd_ref[...] += part

    @pl.when((i == 0) & (j == 0))
    def _():
        cs_ref[...] = jnp.zeros_like(cs_ref)

    cs_ref[...] += jnp.sum(part, axis=0, keepdims=True)


def _neighbor_mean(s_mat, t, m1, x):
    n = x.shape[0]
    blk = 512
    return pl.pallas_call(
        _fd_body,
        grid=(n // blk, n // blk),
        in_specs=[
            pl.BlockSpec((blk, blk), lambda i, j: (i, j)),
            pl.BlockSpec((blk,), lambda i, j: (i,)),
            pl.BlockSpec((blk,), lambda i, j: (i,)),
            pl.BlockSpec((blk, D), lambda i, j: (j, 0)),
        ],
        out_specs=[
            pl.BlockSpec((blk, D), lambda i, j: (i, 0)),
            pl.BlockSpec((1, D), lambda i, j: (0, 0)),
        ],
        out_shape=[jax.ShapeDtypeStruct((n, D), jnp.float32),
                   jax.ShapeDtypeStruct((1, D), jnp.float32)],
    )(s_mat, t, m1, x)


def _sc_gather(table, idx):
    b = idx.shape[0]
    rpt = b // _NW
    mesh = plsc.VectorSubcoreMesh(core_axis_name="c", subcore_axis_name="s")

    @functools.partial(
        pl.kernel, mesh=mesh,
        out_type=jax.ShapeDtypeStruct((b, D), jnp.float32),
        scratch_types=[pltpu.VMEM((rpt,), jnp.int32),
                       pltpu.VMEM((rpt, D), jnp.float32),
                       pltpu.SemaphoreType.DMA])
    def _g(table_hbm, idx_hbm, out_hbm, idx_v, rows_v, sem):
        wid = lax.axis_index("s") * _NC + lax.axis_index("c")
        base = wid * rpt
        pltpu.sync_copy(idx_hbm.at[pl.ds(base, rpt)], idx_v)
        pltpu.async_copy(table_hbm.at[idx_v], rows_v, sem).wait()
        pltpu.sync_copy(rows_v, out_hbm.at[pl.ds(base, rpt)])

    return _g(table, idx)


def _row_cos(a, b):
    num = jnp.sum(a * b, axis=1)
    na = jnp.sqrt(jnp.sum(a * a, axis=1))
    nb = jnp.sqrt(jnp.sum(b * b, axis=1))
    return num / jnp.maximum(na * nb, 1e-8)


def _pred_body(hd_ref, hm_ref, w_ref, src_ref, dst_ref, pred_ref, ls_ref):
    hd = hd_ref[...]
    hm = hm_ref[...]
    p = lax.dot_general(hm, w_ref[...], (((1,), (0,)), ((), ())),
                        preferred_element_type=jnp.float32)
    pred_ref[...] = jax.nn.sigmoid(jnp.sum(hd * p, axis=1))
    pos = _row_cos(hd, src_ref[...])
    neg = _row_cos(hd, dst_ref[...])
    s = jnp.sum(jnp.log(jnp.exp(pos) + jnp.exp(neg)) - pos)

    @pl.when(pl.program_id(0) == 0)
    def _():
        ls_ref[...] = jnp.zeros_like(ls_ref)

    ls_ref[...] += s


def _pred_and_loss(h_d, h_m, w, src, dst):
    b = h_d.shape[0]
    blk = 512
    return pl.pallas_call(
        _pred_body,
        grid=(b // blk,),
        in_specs=[
            pl.BlockSpec((blk, D), lambda i: (i, 0)),
            pl.BlockSpec((blk, D), lambda i: (i, 0)),
            pl.BlockSpec((D, D), lambda i: (0, 0)),
            pl.BlockSpec((blk, D), lambda i: (i, 0)),
            pl.BlockSpec((blk, D), lambda i: (i, 0)),
        ],
        out_specs=[
            pl.BlockSpec((blk,), lambda i: (i,)),
            pl.BlockSpec((1, 128), lambda i: (0, 0)),
        ],
        out_shape=[jax.ShapeDtypeStruct((b,), jnp.float32),
                   jax.ShapeDtypeStruct((1, 128), jnp.float32)],
    )(h_d, h_m, w, src, dst)


def _floss_body(pos_div, neg_div, f_ref, csp_ref, csn_ref, ls_ref):
    f = f_ref[...]
    cp = csp_ref[...] * (1.0 / pos_div)
    cn = csn_ref[...] * (1.0 / neg_div)
    num_p = jnp.sum(f * cp, axis=1)
    num_n = jnp.sum(f * cn, axis=1)
    nf = jnp.sqrt(jnp.sum(f * f, axis=1))
    ncp = jnp.sqrt(jnp.sum(cp * cp))
    ncn = jnp.sqrt(jnp.sum(cn * cn))
    sp = num_p / jnp.maximum(nf * ncp, 1e-8)
    sn = num_n / jnp.maximum(nf * ncn, 1e-8)
    s = jnp.sum(jnp.log(jnp.exp(sp) + jnp.exp(sn)) - sp)

    @pl.when(pl.program_id(0) == 0)
    def _():
        ls_ref[...] = jnp.zeros_like(ls_ref)

    ls_ref[...] += s


def _feature_loss(f, cs_pos, pos_div, cs_neg, neg_div):
    n = f.shape[0]
    blk = 512
    return pl.pallas_call(
        functools.partial(_floss_body, pos_div, neg_div),
        grid=(n // blk,),
        in_specs=[
            pl.BlockSpec((blk, D), lambda i: (i, 0)),
            pl.BlockSpec((1, D), lambda i: (0, 0)),
            pl.BlockSpec((1, D), lambda i: (0, 0)),
        ],
        out_specs=pl.BlockSpec((1, 128), lambda i: (0, 0)),
        out_shape=jax.ShapeDtypeStruct((1, 128), jnp.float32),
    )(f, cs_pos, cs_neg)


def kernel(emb, h, src_init, dst_init, W, diseases, mirnas):
    xd = emb[:N_DISEASES]
    xm = emb[N_DISEASES:]

    s1 = _similarity(xd)
    s2 = _similarity(xm)
    res1 = _sc_select(s1, _chunk_max(s1))
    res2 = _sc_select(s2, _chunk_max(s2))
    m1_d, t_d = res1[:, 0], res1[:, 1]
    m1_m, t_m = res2[:, 0], res2[:, 1]
    fd, cs_d = _neighbor_mean(s1, t_d, m1_d, xd)
    fm, cs_m = _neighbor_mean(s2, t_m, m1_m, xm)

    h_d = _sc_gather(h, diseases.astype(jnp.int32))
    h_m = _sc_gather(h, mirnas.astype(jnp.int32))

    pred, closs_parts = _pred_and_loss(h_d, h_m, W, src_init, dst_init)
    contrastive_loss = closs_parts[0, 0] / h_d.shape[0]

    nd = float(fd.shape[0])
    nm = float(fm.shape[0])
    ld_parts = _feature_loss(fd, cs_d, nd, cs_m, nm)
    lm_parts = _feature_loss(fm, cs_m, nm, cs_d, nd)
    feature_contrastive_loss = (ld_parts[0, 0] / nd +
                                lm_parts[0, 0] / nm) / 2.0

    return (pred, contrastive_loss, feature_contrastive_loss)


# speedup vs baseline: 1.4362x; 1.4362x over previous
import functools

import numpy as np

import jax
import jax.numpy as jnp
from jax import lax
from jax.experimental import pallas as pl
from jax.experimental.pallas import tpu as pltpu
from jax.experimental.pallas import tpu_sc as plsc

N_DISEASES = 2048
K = 50
D = 512
_NC, _NS = 2, 16
_NW = _NC * _NS
_NEG = np.float32(-np.inf)


def _sim_body(xr_ref, xc_ref, o_ref):
    o_ref[...] = lax.dot_general(
        xr_ref[...], xc_ref[...], (((1,), (1,)), ((), ())),
        preferred_element_type=jnp.float32)


def _similarity(x):
    n = x.shape[0]
    blk = 512
    return pl.pallas_call(
        _sim_body,
        grid=(n // blk, n // blk),
        in_specs=[
            pl.BlockSpec((blk, D), lambda i, j: (i, 0)),
            pl.BlockSpec((blk, D), lambda i, j: (j, 0)),
        ],
        out_specs=pl.BlockSpec((blk, blk), lambda i, j: (i, j)),
        out_shape=jax.ShapeDtypeStruct((n, n), jnp.float32),
    )(x, x)


def _s16(x):
    return plsc.sort_key_val(x, x, descending=True)[0]


def _rev(x):
    return jnp.flip(x, 0)


def _merge2(a, b):
    rb = _rev(b)
    return _s16(jnp.maximum(a, rb)), _s16(jnp.minimum(a, rb))


def _bm32(x0, x1):
    return _s16(jnp.maximum(x0, x1)), _s16(jnp.minimum(x0, x1))


def _merge4(a, b):
    rb0, rb1 = _rev(b[1]), _rev(b[0])
    h0, h1 = jnp.maximum(a[0], rb0), jnp.maximum(a[1], rb1)
    l0, l1 = jnp.minimum(a[0], rb0), jnp.minimum(a[1], rb1)
    return _bm32(h0, h1) + _bm32(l0, l1)


def _sort64(r):
    s = [_s16(x) for x in r]
    return _merge4(_merge2(s[0], s[1]), _merge2(s[2], s[3]))


def _merge8_top4(a, b):
    rb = [_rev(b[3]), _rev(b[2]), _rev(b[1]), _rev(b[0])]
    h = [jnp.maximum(a[j], rb[j]) for j in range(4)]
    p0, p1 = jnp.maximum(h[0], h[2]), jnp.maximum(h[1], h[3])
    q0, q1 = jnp.minimum(h[0], h[2]), jnp.minimum(h[1], h[3])
    return _bm32(p0, p1) + _bm32(q0, q1)


def _top64(regs):
    t = _sort64(regs[0:4])
    for base in range(4, len(regs), 4):
        t = _merge8_top4(t, _sort64(regs[base:base + 4]))
    return t


def _sc_select(s_mat):
    m, n = s_mat.shape
    rpt = m // _NW
    nch = n // 16
    mesh = plsc.VectorSubcoreMesh(core_axis_name="c", subcore_axis_name="s")

    STR = 32
    BUF = 16 * STR

    @functools.partial(
        pl.kernel, mesh=mesh,
        out_type=jax.ShapeDtypeStruct((m, 16), jnp.float32),
        compiler_params=pltpu.CompilerParams(needs_layout_passes=False),
        scratch_types=[pltpu.VMEM((2, n), jnp.float32),
                       pltpu.VMEM((BUF,), jnp.float32),
                       pltpu.VMEM((192,), jnp.float32),
                       pltpu.VMEM((rpt, 16), jnp.float32),
                       pltpu.SemaphoreType.DMA,
                       pltpu.SemaphoreType.DMA])
    def _sel(s_hbm, res_hbm, rowbuf, buf, lst, resv, sem0, sem1):
        wid = lax.axis_index("s") * _NC + lax.axis_index("c")
        row0 = wid * rpt
        iota = lax.iota(jnp.int32, 16)
        sems = (sem0, sem1)
        neg16 = jnp.full((16,), _NEG, jnp.float32)

        def bfly(v, op):
            for s in (8, 4, 2, 1):
                v = op(v, v[iota ^ s])
            return v

        def ins4(L, v):
            out = []
            for j in range(4):
                out.append(jnp.maximum(L[j], v))
                v = jnp.minimum(L[j], v)
            return out

        def top64_of_list(off):
            regs = []
            for j in range(12):
                v = lst[pl.ds(j * 16, 16)]
                regs.append(jnp.where(iota + j * 16 < off, v, _NEG))
            return _top64(regs)

        def lane(vec, k):
            return jnp.max(jnp.where(iota == k, vec, _NEG))

        def fallback_t(b):
            def chunk2(g, carry):
                theta, off = carry
                for u in range(2):
                    v = rowbuf[b, pl.ds((2 * g + u) * 16, 16)]
                    msk = v > theta
                    sk = plsc.sort_key_val(v, v, mask=msk,
                                           descending=True)[0]
                    lst[pl.ds(off, 16)] = sk
                    off = off + jnp.sum(msk.astype(jnp.int32))

                def compact(th2, of2):
                    top = top64_of_list(of2)
                    for j in range(4):
                        lst[pl.ds(j * 16, 16)] = top[j]
                    return lane(top[3], 2), np.int32(64)

                return lax.cond(off > 128, compact,
                                lambda th2, of2: (th2, of2), theta, off)

            theta, off = lax.fori_loop(0, nch // 2, chunk2,
                                       (_NEG, np.int32(0)))
            top = top64_of_list(off)
            return jnp.maximum(lane(top[3], 2), theta)

        def process(b, row_i):
            def step_a(g, carry):
                la, lb = carry
                pa = jnp.maximum(rowbuf[b, pl.ds((4 * g) * 16, 16)],
                                 rowbuf[b, pl.ds((4 * g + 1) * 16, 16)])
                pb = jnp.maximum(rowbuf[b, pl.ds((4 * g + 2) * 16, 16)],
                                 rowbuf[b, pl.ds((4 * g + 3) * 16, 16)])
                return tuple(ins4(list(la), pa)), tuple(ins4(list(lb), pb))

            la, lb = lax.fori_loop(0, nch // 4, step_a,
                                   ((neg16,) * 4, (neg16,) * 4), unroll=2)
            top4 = list(la)
            for r in lb:
                top4 = ins4(top4, r)
            th = bfly(top4[3], jnp.minimum)
            m1s = bfly(top4[0], jnp.maximum)

            for j in range(BUF // 16):
                buf[pl.ds(j * 16, 16)] = neg16

            def step_b(i, cnt):
                v = rowbuf[b, pl.ds(i * 16, 16)]
                msk = (v > th) & (cnt < STR)
                idx = cnt * 16 + iota
                plsc.store_scatter(buf, [idx], v, mask=msk)
                return cnt + msk.astype(jnp.int32)

            cnt = lax.fori_loop(0, nch, step_b,
                                jnp.zeros((16,), jnp.int32), unroll=4)
            c_tot = bfly(cnt, jnp.add)[0]
            ov = bfly(cnt, jnp.maximum)[0]

            def sel_from(nreg):
                def f():
                    regs = [buf[pl.ds(j * 16, 16)] for j in range(nreg)]
                    return lane(_top64(regs)[3], 2)
                return f

            def t_main():
                return lax.cond(
                    c_tot > 50,
                    lambda: lax.cond(ov <= 8, sel_from(8),
                                     lambda: lax.cond(ov <= 16, sel_from(16),
                                                      sel_from(STR))),
                    lambda: th[0])

            t = lax.cond(ov >= STR, lambda: fallback_t(b), t_main)
            res = jnp.where(iota == 0, m1s, jnp.where(iota == 1, t, 0.0))
            resv[row_i, :] = res

        def start(r, b):
            pltpu.async_copy(s_hbm.at[row0 + r], rowbuf.at[b], sems[b])

        def wait(b):
            pltpu.make_async_copy(s_hbm.at[0], rowbuf.at[b], sems[b]).wait()

        start(0, 0)

        def pair(g, _):
            r = 2 * g
            start(r + 1, 1)
            wait(0)
            process(0, r)

            @pl.when(r + 2 < rpt)
            def _():
                start(r + 2, 0)

            wait(1)
            process(1, r + 1)
            return 0

        lax.fori_loop(0, rpt // 2, pair, 0)
        pltpu.sync_copy(resv, res_hbm.at[pl.ds(row0, rpt)])

    return _sel(s_mat)


def _fd_body(s_ref, t_ref, m1_ref, x_ref, fd_ref, cs_ref):
    i, j = pl.program_id(0), pl.program_id(1)
    s = s_ref[...]
    t = t_ref[...].reshape(-1, 1)
    m1 = m1_ref[...].reshape(-1, 1)
    msk = ((s >= t) & (s != m1)).astype(jnp.float32)
    part = lax.dot_general(msk, x_ref[...], (((1,), (0,)), ((), ())),
                           preferred_element_type=jnp.float32) * (1.0 / K)

    @pl.when(j == 0)
    def _():
        fd_ref[...] = jnp.zeros_like(fd_ref)

    fd_ref[...] += part

    @pl.when((i == 0) & (j == 0))
    def _():
        cs_ref[...] = jnp.zeros_like(cs_ref)

    cs_ref[...] += jnp.sum(part, axis=0, keepdims=True)


def _neighbor_mean(s_mat, t, m1, x):
    n = x.shape[0]
    blk = 512
    return pl.pallas_call(
        _fd_body,
        grid=(n // blk, n // blk),
        in_specs=[
            pl.BlockSpec((blk, blk), lambda i, j: (i, j)),
            pl.BlockSpec((blk,), lambda i, j: (i,)),
            pl.BlockSpec((blk,), lambda i, j: (i,)),
            pl.BlockSpec((blk, D), lambda i, j: (j, 0)),
        ],
        out_specs=[
            pl.BlockSpec((blk, D), lambda i, j: (i, 0)),
            pl.BlockSpec((1, D), lambda i, j: (0, 0)),
        ],
        out_shape=[jax.ShapeDtypeStruct((n, D), jnp.float32),
                   jax.ShapeDtypeStruct((1, D), jnp.float32)],
    )(s_mat, t, m1, x)


def _sc_gather(table, idx):
    b = idx.shape[0]
    rpt = b // _NW
    mesh = plsc.VectorSubcoreMesh(core_axis_name="c", subcore_axis_name="s")

    @functools.partial(
        pl.kernel, mesh=mesh,
        out_type=jax.ShapeDtypeStruct((b, D), jnp.float32),
        scratch_types=[pltpu.VMEM((rpt,), jnp.int32),
                       pltpu.VMEM((rpt, D), jnp.float32),
                       pltpu.SemaphoreType.DMA])
    def _g(table_hbm, idx_hbm, out_hbm, idx_v, rows_v, sem):
        wid = lax.axis_index("s") * _NC + lax.axis_index("c")
        base = wid * rpt
        pltpu.sync_copy(idx_hbm.at[pl.ds(base, rpt)], idx_v)
        pltpu.async_copy(table_hbm.at[idx_v], rows_v, sem).wait()
        pltpu.sync_copy(rows_v, out_hbm.at[pl.ds(base, rpt)])

    return _g(table, idx)


def _row_cos(a, b):
    num = jnp.sum(a * b, axis=1)
    na = jnp.sqrt(jnp.sum(a * a, axis=1))
    nb = jnp.sqrt(jnp.sum(b * b, axis=1))
    return num / jnp.maximum(na * nb, 1e-8)


def _pred_body(hd_ref, hm_ref, w_ref, src_ref, dst_ref, pred_ref, ls_ref):
    hd = hd_ref[...]
    hm = hm_ref[...]
    p = lax.dot_general(hm, w_ref[...], (((1,), (0,)), ((), ())),
                        preferred_element_type=jnp.float32)
    pred_ref[...] = jax.nn.sigmoid(jnp.sum(hd * p, axis=1))
    pos = _row_cos(hd, src_ref[...])
    neg = _row_cos(hd, dst_ref[...])
    s = jnp.sum(jnp.log(jnp.exp(pos) + jnp.exp(neg)) - pos)

    @pl.when(pl.program_id(0) == 0)
    def _():
        ls_ref[...] = jnp.zeros_like(ls_ref)

    ls_ref[...] += s


def _pred_and_loss(h_d, h_m, w, src, dst):
    b = h_d.shape[0]
    blk = 512
    return pl.pallas_call(
        _pred_body,
        grid=(b // blk,),
        in_specs=[
            pl.BlockSpec((blk, D), lambda i: (i, 0)),
            pl.BlockSpec((blk, D), lambda i: (i, 0)),
            pl.BlockSpec((D, D), lambda i: (0, 0)),
            pl.BlockSpec((blk, D), lambda i: (i, 0)),
            pl.BlockSpec((blk, D), lambda i: (i, 0)),
        ],
        out_specs=[
            pl.BlockSpec((blk,), lambda i: (i,)),
            pl.BlockSpec((1, 128), lambda i: (0, 0)),
        ],
        out_shape=[jax.ShapeDtypeStruct((b,), jnp.float32),
                   jax.ShapeDtypeStruct((1, 128), jnp.float32)],
    )(h_d, h_m, w, src, dst)


def _floss_body(pos_div, neg_div, f_ref, csp_ref, csn_ref, ls_ref):
    f = f_ref[...]
    cp = csp_ref[...] * (1.0 / pos_div)
    cn = csn_ref[...] * (1.0 / neg_div)
    num_p = jnp.sum(f * cp, axis=1)
    num_n = jnp.sum(f * cn, axis=1)
    nf = jnp.sqrt(jnp.sum(f * f, axis=1))
    ncp = jnp.sqrt(jnp.sum(cp * cp))
    ncn = jnp.sqrt(jnp.sum(cn * cn))
    sp = num_p / jnp.maximum(nf * ncp, 1e-8)
    sn = num_n / jnp.maximum(nf * ncn, 1e-8)
    s = jnp.sum(jnp.log(jnp.exp(sp) + jnp.exp(sn)) - sp)

    @pl.when(pl.program_id(0) == 0)
    def _():
        ls_ref[...] = jnp.zeros_like(ls_ref)

    ls_ref[...] += s


def _feature_loss(f, cs_pos, pos_div, cs_neg, neg_div):
    n = f.shape[0]
    blk = 512
    return pl.pallas_call(
        functools.partial(_floss_body, pos_div, neg_div),
        grid=(n // blk,),
        in_specs=[
            pl.BlockSpec((blk, D), lambda i: (i, 0)),
            pl.BlockSpec((1, D), lambda i: (0, 0)),
            pl.BlockSpec((1, D), lambda i: (0, 0)),
        ],
        out_specs=pl.BlockSpec((1, 128), lambda i: (0, 0)),
        out_shape=jax.ShapeDtypeStruct((1, 128), jnp.float32),
    )(f, cs_pos, cs_neg)


def kernel(emb, h, src_init, dst_init, W, diseases, mirnas):
    xd = emb[:N_DISEASES]
    xm = emb[N_DISEASES:]

    s1 = _similarity(xd)
    s2 = _similarity(xm)
    res1 = _sc_select(s1)
    res2 = _sc_select(s2)
    m1_d, t_d = res1[:, 0], res1[:, 1]
    m1_m, t_m = res2[:, 0], res2[:, 1]
    fd, cs_d = _neighbor_mean(s1, t_d, m1_d, xd)
    fm, cs_m = _neighbor_mean(s2, t_m, m1_m, xm)

    h_d = _sc_gather(h, diseases.astype(jnp.int32))
    h_m = _sc_gather(h, mirnas.astype(jnp.int32))

    pred, closs_parts = _pred_and_loss(h_d, h_m, W, src_init, dst_init)
    contrastive_loss = closs_parts[0, 0] / h_d.shape[0]

    nd = float(fd.shape[0])
    nm = float(fm.shape[0])
    ld_parts = _feature_loss(fd, cs_d, nd, cs_m, nm)
    lm_parts = _feature_loss(fm, cs_m, nm, cs_d, nd)
    feature_contrastive_loss = (ld_parts[0, 0] / nd +
                                lm_parts[0, 0] / nm) / 2.0

    return (pred, contrastive_loss, feature_contrastive_loss)


# speedup vs baseline: 1.4557x; 1.0136x over previous
import functools

import numpy as np

import jax
import jax.numpy as jnp
from jax import lax
from jax.experimental import pallas as pl
from jax.experimental.pallas import tpu as pltpu
from jax.experimental.pallas import tpu_sc as plsc

N_DISEASES = 2048
K = 50
D = 512
_NC, _NS = 2, 16
_NW = _NC * _NS
_NEG = np.float32(-np.inf)


def _sim_body(xr_ref, xc_ref, o_ref):
    o_ref[...] = lax.dot_general(
        xr_ref[...], xc_ref[...], (((1,), (1,)), ((), ())),
        preferred_element_type=jnp.float32)


def _similarity(x):
    n = x.shape[0]
    blk = 512
    return pl.pallas_call(
        _sim_body,
        grid=(n // blk, n // blk),
        in_specs=[
            pl.BlockSpec((blk, D), lambda i, j: (i, 0)),
            pl.BlockSpec((blk, D), lambda i, j: (j, 0)),
        ],
        out_specs=pl.BlockSpec((blk, blk), lambda i, j: (i, j)),
        out_shape=jax.ShapeDtypeStruct((n, n), jnp.float32),
    )(x, x)


def _s16(x):
    return plsc.sort_key_val(x, x, descending=True)[0]


def _rev(x):
    return jnp.flip(x, 0)


def _merge2(a, b):
    rb = _rev(b)
    return _s16(jnp.maximum(a, rb)), _s16(jnp.minimum(a, rb))


def _bm32(x0, x1):
    return _s16(jnp.maximum(x0, x1)), _s16(jnp.minimum(x0, x1))


def _merge4(a, b):
    rb0, rb1 = _rev(b[1]), _rev(b[0])
    h0, h1 = jnp.maximum(a[0], rb0), jnp.maximum(a[1], rb1)
    l0, l1 = jnp.minimum(a[0], rb0), jnp.minimum(a[1], rb1)
    return _bm32(h0, h1) + _bm32(l0, l1)


def _sort64(r):
    s = [_s16(x) for x in r]
    return _merge4(_merge2(s[0], s[1]), _merge2(s[2], s[3]))


def _merge8_top4(a, b):
    rb = [_rev(b[3]), _rev(b[2]), _rev(b[1]), _rev(b[0])]
    h = [jnp.maximum(a[j], rb[j]) for j in range(4)]
    p0, p1 = jnp.maximum(h[0], h[2]), jnp.maximum(h[1], h[3])
    q0, q1 = jnp.minimum(h[0], h[2]), jnp.minimum(h[1], h[3])
    return _bm32(p0, p1) + _bm32(q0, q1)


def _top64(regs):
    t = _sort64(regs[0:4])
    for base in range(4, len(regs), 4):
        t = _merge8_top4(t, _sort64(regs[base:base + 4]))
    return t


def _sc_select(s_mat):
    m, n = s_mat.shape
    rpt = m // _NW
    nch = n // 16
    mesh = plsc.VectorSubcoreMesh(core_axis_name="c", subcore_axis_name="s")

    STR = 32
    BUF = 16 * STR

    @functools.partial(
        pl.kernel, mesh=mesh,
        out_type=jax.ShapeDtypeStruct((m, 16), jnp.float32),
        compiler_params=pltpu.CompilerParams(needs_layout_passes=False),
        scratch_types=[pltpu.VMEM((2, n), jnp.float32),
                       pltpu.VMEM((BUF,), jnp.float32),
                       pltpu.VMEM((192,), jnp.float32),
                       pltpu.VMEM((rpt, 16), jnp.float32),
                       pltpu.SemaphoreType.DMA,
                       pltpu.SemaphoreType.DMA])
    def _sel(s_hbm, res_hbm, rowbuf, buf, lst, resv, sem0, sem1):
        wid = lax.axis_index("s") * _NC + lax.axis_index("c")
        row0 = wid * rpt
        iota = lax.iota(jnp.int32, 16)
        sems = (sem0, sem1)
        neg16 = jnp.full((16,), _NEG, jnp.float32)

        def bfly(v, op):
            for s in (8, 4, 2, 1):
                v = op(v, v[iota ^ s])
            return v

        def ins4(L, v):
            out = []
            for j in range(4):
                out.append(jnp.maximum(L[j], v))
                v = jnp.minimum(L[j], v)
            return out

        def top64_of_list(off):
            regs = []
            for j in range(12):
                v = lst[pl.ds(j * 16, 16)]
                regs.append(jnp.where(iota + j * 16 < off, v, _NEG))
            return _top64(regs)

        def lane(vec, k):
            return jnp.max(jnp.where(iota == k, vec, _NEG))

        def fallback_t(b):
            def chunk2(g, carry):
                theta, off = carry
                for u in range(2):
                    v = rowbuf[b, pl.ds((2 * g + u) * 16, 16)]
                    msk = v > theta
                    sk = plsc.sort_key_val(v, v, mask=msk,
                                           descending=True)[0]
                    lst[pl.ds(off, 16)] = sk
                    off = off + jnp.sum(msk.astype(jnp.int32))

                def compact(th2, of2):
                    top = top64_of_list(of2)
                    for j in range(4):
                        lst[pl.ds(j * 16, 16)] = top[j]
                    return lane(top[3], 2), np.int32(64)

                return lax.cond(off > 128, compact,
                                lambda th2, of2: (th2, of2), theta, off)

            theta, off = lax.fori_loop(0, nch // 2, chunk2,
                                       (_NEG, np.int32(0)))
            top = top64_of_list(off)
            return jnp.maximum(lane(top[3], 2), theta)

        def process(b, row_i):
            def step_a(g, carry):
                la, lb = carry

                def quad(c0):
                    va = jnp.maximum(rowbuf[b, pl.ds(c0 * 16, 16)],
                                     rowbuf[b, pl.ds((c0 + 1) * 16, 16)])
                    vb = jnp.maximum(rowbuf[b, pl.ds((c0 + 2) * 16, 16)],
                                     rowbuf[b, pl.ds((c0 + 3) * 16, 16)])
                    return jnp.maximum(va, vb)

                return (tuple(ins4(list(la), quad(8 * g))),
                        tuple(ins4(list(lb), quad(8 * g + 4))))

            la, lb = lax.fori_loop(0, nch // 8, step_a,
                                   ((neg16,) * 4, (neg16,) * 4), unroll=2)
            top4 = list(la)
            for r in lb:
                top4 = ins4(top4, r)
            th = bfly(top4[3], jnp.minimum)
            m1s = bfly(top4[0], jnp.maximum)

            for j in range(BUF // 16):
                buf[pl.ds(j * 16, 16)] = neg16

            def step_b(i, addr):
                v = rowbuf[b, pl.ds(i * 16, 16)]
                msk = (v > th) & (addr < BUF)
                plsc.store_scatter(buf, [addr], v, mask=msk)
                return addr + jnp.where(msk, np.int32(16), np.int32(0))

            addr = lax.fori_loop(0, nch, step_b, iota, unroll=4)
            cnt = lax.shift_right_logical(addr - iota, 4)
            c_tot = bfly(cnt, jnp.add)[0]
            ov = bfly(cnt, jnp.maximum)[0]

            def sel_from(nreg):
                def f():
                    regs = [buf[pl.ds(j * 16, 16)] for j in range(nreg)]
                    return lane(_top64(regs)[3], 2)
                return f

            def t_main():
                return lax.cond(
                    c_tot > 50,
                    lambda: lax.cond(ov <= 8, sel_from(8),
                                     lambda: lax.cond(ov <= 16, sel_from(16),
                                                      sel_from(STR))),
                    lambda: th[0])

            t = lax.cond(ov >= STR, lambda: fallback_t(b), t_main)
            res = jnp.where(iota == 0, m1s, jnp.where(iota == 1, t, 0.0))
            resv[row_i, :] = res

        def start(r, b):
            pltpu.async_copy(s_hbm.at[row0 + r], rowbuf.at[b], sems[b])

        def wait(b):
            pltpu.make_async_copy(s_hbm.at[0], rowbuf.at[b], sems[b]).wait()

        start(0, 0)

        def pair(g, _):
            r = 2 * g
            start(r + 1, 1)
            wait(0)
            process(0, r)

            @pl.when(r + 2 < rpt)
            def _():
                start(r + 2, 0)

            wait(1)
            process(1, r + 1)
            return 0

        lax.fori_loop(0, rpt // 2, pair, 0)
        pltpu.sync_copy(resv, res_hbm.at[pl.ds(row0, rpt)])

    return _sel(s_mat)


def _fd_body(s_ref, t_ref, m1_ref, x_ref, fd_ref, cs_ref):
    i, j = pl.program_id(0), pl.program_id(1)
    s = s_ref[...]
    t = t_ref[...].reshape(-1, 1)
    m1 = m1_ref[...].reshape(-1, 1)
    msk = ((s >= t) & (s != m1)).astype(jnp.float32)
    part = lax.dot_general(msk, x_ref[...], (((1,), (0,)), ((), ())),
                           preferred_element_type=jnp.float32) * (1.0 / K)

    @pl.when(j == 0)
    def _():
        fd_ref[...] = jnp.zeros_like(fd_ref)

    fd_ref[...] += part

    @pl.when((i == 0) & (j == 0))
    def _():
        cs_ref[...] = jnp.zeros_like(cs_ref)

    cs_ref[...] += jnp.sum(part, axis=0, keepdims=True)


def _neighbor_mean(s_mat, t, m1, x):
    n = x.shape[0]
    blk = 512
    return pl.pallas_call(
        _fd_body,
        grid=(n // blk, n // blk),
        in_specs=[
            pl.BlockSpec((blk, blk), lambda i, j: (i, j)),
            pl.BlockSpec((blk,), lambda i, j: (i,)),
            pl.BlockSpec((blk,), lambda i, j: (i,)),
            pl.BlockSpec((blk, D), lambda i, j: (j, 0)),
        ],
        out_specs=[
            pl.BlockSpec((blk, D), lambda i, j: (i, 0)),
            pl.BlockSpec((1, D), lambda i, j: (0, 0)),
        ],
        out_shape=[jax.ShapeDtypeStruct((n, D), jnp.float32),
                   jax.ShapeDtypeStruct((1, D), jnp.float32)],
    )(s_mat, t, m1, x)


def _sc_gather(table, idx):
    b = idx.shape[0]
    rpt = b // _NW
    mesh = plsc.VectorSubcoreMesh(core_axis_name="c", subcore_axis_name="s")

    @functools.partial(
        pl.kernel, mesh=mesh,
        out_type=jax.ShapeDtypeStruct((b, D), jnp.float32),
        scratch_types=[pltpu.VMEM((rpt,), jnp.int32),
                       pltpu.VMEM((rpt, D), jnp.float32),
                       pltpu.SemaphoreType.DMA])
    def _g(table_hbm, idx_hbm, out_hbm, idx_v, rows_v, sem):
        wid = lax.axis_index("s") * _NC + lax.axis_index("c")
        base = wid * rpt
        pltpu.sync_copy(idx_hbm.at[pl.ds(base, rpt)], idx_v)
        pltpu.async_copy(table_hbm.at[idx_v], rows_v, sem).wait()
        pltpu.sync_copy(rows_v, out_hbm.at[pl.ds(base, rpt)])

    return _g(table, idx)


def _row_cos(a, b):
    num = jnp.sum(a * b, axis=1)
    na = jnp.sqrt(jnp.sum(a * a, axis=1))
    nb = jnp.sqrt(jnp.sum(b * b, axis=1))
    return num / jnp.maximum(na * nb, 1e-8)


def _pred_body(hd_ref, hm_ref, w_ref, src_ref, dst_ref, pred_ref, ls_ref):
    hd = hd_ref[...]
    hm = hm_ref[...]
    p = lax.dot_general(hm, w_ref[...], (((1,), (0,)), ((), ())),
                        preferred_element_type=jnp.float32)
    pred_ref[...] = jax.nn.sigmoid(jnp.sum(hd * p, axis=1))
    pos = _row_cos(hd, src_ref[...])
    neg = _row_cos(hd, dst_ref[...])
    s = jnp.sum(jnp.log(jnp.exp(pos) + jnp.exp(neg)) - pos)

    @pl.when(pl.program_id(0) == 0)
    def _():
        ls_ref[...] = jnp.zeros_like(ls_ref)

    ls_ref[...] += s


def _pred_and_loss(h_d, h_m, w, src, dst):
    b = h_d.shape[0]
    blk = 512
    return pl.pallas_call(
        _pred_body,
        grid=(b // blk,),
        in_specs=[
            pl.BlockSpec((blk, D), lambda i: (i, 0)),
            pl.BlockSpec((blk, D), lambda i: (i, 0)),
            pl.BlockSpec((D, D), lambda i: (0, 0)),
            pl.BlockSpec((blk, D), lambda i: (i, 0)),
            pl.BlockSpec((blk, D), lambda i: (i, 0)),
        ],
        out_specs=[
            pl.BlockSpec((blk,), lambda i: (i,)),
            pl.BlockSpec((1, 128), lambda i: (0, 0)),
        ],
        out_shape=[jax.ShapeDtypeStruct((b,), jnp.float32),
                   jax.ShapeDtypeStruct((1, 128), jnp.float32)],
    )(h_d, h_m, w, src, dst)


def _floss_body(pos_div, neg_div, f_ref, csp_ref, csn_ref, ls_ref):
    f = f_ref[...]
    cp = csp_ref[...] * (1.0 / pos_div)
    cn = csn_ref[...] * (1.0 / neg_div)
    num_p = jnp.sum(f * cp, axis=1)
    num_n = jnp.sum(f * cn, axis=1)
    nf = jnp.sqrt(jnp.sum(f * f, axis=1))
    ncp = jnp.sqrt(jnp.sum(cp * cp))
    ncn = jnp.sqrt(jnp.sum(cn * cn))
    sp = num_p / jnp.maximum(nf * ncp, 1e-8)
    sn = num_n / jnp.maximum(nf * ncn, 1e-8)
    s = jnp.sum(jnp.log(jnp.exp(sp) + jnp.exp(sn)) - sp)

    @pl.when(pl.program_id(0) == 0)
    def _():
        ls_ref[...] = jnp.zeros_like(ls_ref)

    ls_ref[...] += s


def _feature_loss(f, cs_pos, pos_div, cs_neg, neg_div):
    n = f.shape[0]
    blk = 512
    return pl.pallas_call(
        functools.partial(_floss_body, pos_div, neg_div),
        grid=(n // blk,),
        in_specs=[
            pl.BlockSpec((blk, D), lambda i: (i, 0)),
            pl.BlockSpec((1, D), lambda i: (0, 0)),
            pl.BlockSpec((1, D), lambda i: (0, 0)),
        ],
        out_specs=pl.BlockSpec((1, 128), lambda i: (0, 0)),
        out_shape=jax.ShapeDtypeStruct((1, 128), jnp.float32),
    )(f, cs_pos, cs_neg)


def kernel(emb, h, src_init, dst_init, W, diseases, mirnas):
    xd = emb[:N_DISEASES]
    xm = emb[N_DISEASES:]

    s1 = _similarity(xd)
    s2 = _similarity(xm)
    res1 = _sc_select(s1)
    res2 = _sc_select(s2)
    m1_d, t_d = res1[:, 0], res1[:, 1]
    m1_m, t_m = res2[:, 0], res2[:, 1]
    fd, cs_d = _neighbor_mean(s1, t_d, m1_d, xd)
    fm, cs_m = _neighbor_mean(s2, t_m, m1_m, xm)

    h_d = _sc_gather(h, diseases.astype(jnp.int32))
    h_m = _sc_gather(h, mirnas.astype(jnp.int32))

    pred, closs_parts = _pred_and_loss(h_d, h_m, W, src_init, dst_init)
    contrastive_loss = closs_parts[0, 0] / h_d.shape[0]

    nd = float(fd.shape[0])
    nm = float(fm.shape[0])
    ld_parts = _feature_loss(fd, cs_d, nd, cs_m, nm)
    lm_parts = _feature_loss(fm, cs_m, nm, cs_d, nd)
    feature_contrastive_loss = (ld_parts[0, 0] / nd +
                                lm_parts[0, 0] / nm) / 2.0

    return (pred, contrastive_loss, feature_contrastive_loss)


# speedup vs baseline: 1.4971x; 1.0284x over previous
import functools

import numpy as np

import jax
import jax.numpy as jnp
from jax import lax
from jax.experimental import pallas as pl
from jax.experimental.pallas import tpu as pltpu
from jax.experimental.pallas import tpu_sc as plsc

N_DISEASES = 2048
K = 50
D = 512
_NC, _NS = 2, 16
_NW = _NC * _NS
_NEG = np.float32(-np.inf)


def _sim_body(xr_ref, xc_ref, o_ref):
    o_ref[...] = lax.dot_general(
        xr_ref[...], xc_ref[...], (((1,), (1,)), ((), ())),
        preferred_element_type=jnp.float32)


def _similarity(x):
    n = x.shape[0]
    blk = 512
    return pl.pallas_call(
        _sim_body,
        grid=(n // blk, n // blk),
        in_specs=[
            pl.BlockSpec((blk, D), lambda i, j: (i, 0)),
            pl.BlockSpec((blk, D), lambda i, j: (j, 0)),
        ],
        out_specs=pl.BlockSpec((blk, blk), lambda i, j: (i, j)),
        out_shape=jax.ShapeDtypeStruct((n, n), jnp.float32),
    )(x, x)


def _s16(x):
    return plsc.sort_key_val(x, x, descending=True)[0]


def _rev(x):
    return jnp.flip(x, 0)


def _merge2(a, b):
    rb = _rev(b)
    return _s16(jnp.maximum(a, rb)), _s16(jnp.minimum(a, rb))


def _bm32(x0, x1):
    return _s16(jnp.maximum(x0, x1)), _s16(jnp.minimum(x0, x1))


def _merge4(a, b):
    rb0, rb1 = _rev(b[1]), _rev(b[0])
    h0, h1 = jnp.maximum(a[0], rb0), jnp.maximum(a[1], rb1)
    l0, l1 = jnp.minimum(a[0], rb0), jnp.minimum(a[1], rb1)
    return _bm32(h0, h1) + _bm32(l0, l1)


def _sort64(r):
    s = [_s16(x) for x in r]
    return _merge4(_merge2(s[0], s[1]), _merge2(s[2], s[3]))


def _merge8_top4(a, b):
    rb = [_rev(b[3]), _rev(b[2]), _rev(b[1]), _rev(b[0])]
    h = [jnp.maximum(a[j], rb[j]) for j in range(4)]
    p0, p1 = jnp.maximum(h[0], h[2]), jnp.maximum(h[1], h[3])
    q0, q1 = jnp.minimum(h[0], h[2]), jnp.minimum(h[1], h[3])
    return _bm32(p0, p1) + _bm32(q0, q1)


def _top64(regs):
    t = _sort64(regs[0:4])
    for base in range(4, len(regs), 4):
        t = _merge8_top4(t, _sort64(regs[base:base + 4]))
    return t


def _sc_select(s_mat):
    m, n = s_mat.shape
    rpt = m // _NW
    nch = n // 16
    mesh = plsc.VectorSubcoreMesh(core_axis_name="c", subcore_axis_name="s")

    STR = 32
    BUF = 16 * STR

    @functools.partial(
        pl.kernel, mesh=mesh,
        out_type=jax.ShapeDtypeStruct((m, 16), jnp.float32),
        compiler_params=pltpu.CompilerParams(needs_layout_passes=False),
        scratch_types=[pltpu.VMEM((2, n), jnp.float32),
                       pltpu.VMEM((BUF,), jnp.float32),
                       pltpu.VMEM((192,), jnp.float32),
                       pltpu.VMEM((rpt, 16), jnp.float32),
                       pltpu.SemaphoreType.DMA,
                       pltpu.SemaphoreType.DMA])
    def _sel(s_hbm, res_hbm, rowbuf, buf, lst, resv, sem0, sem1):
        wid = lax.axis_index("s") * _NC + lax.axis_index("c")
        row0 = wid * rpt
        iota = lax.iota(jnp.int32, 16)
        sems = (sem0, sem1)
        neg16 = jnp.full((16,), _NEG, jnp.float32)

        def bfly(v, op):
            for s in (8, 4, 2, 1):
                v = op(v, v[iota ^ s])
            return v

        def ins4(L, v):
            out = []
            for j in range(4):
                out.append(jnp.maximum(L[j], v))
                v = jnp.minimum(L[j], v)
            return out

        def top64_of_list(off):
            regs = []
            for j in range(12):
                v = lst[pl.ds(j * 16, 16)]
                regs.append(jnp.where(iota + j * 16 < off, v, _NEG))
            return _top64(regs)

        def lane(vec, k):
            return jnp.max(jnp.where(iota == k, vec, _NEG))

        def fallback_t(b):
            def chunk2(g, carry):
                theta, off = carry
                for u in range(2):
                    v = rowbuf[b, pl.ds((2 * g + u) * 16, 16)]
                    msk = v > theta
                    sk = plsc.sort_key_val(v, v, mask=msk,
                                           descending=True)[0]
                    lst[pl.ds(off, 16)] = sk
                    off = off + jnp.sum(msk.astype(jnp.int32))

                def compact(th2, of2):
                    top = top64_of_list(of2)
                    for j in range(4):
                        lst[pl.ds(j * 16, 16)] = top[j]
                    return lane(top[3], 2), np.int32(64)

                return lax.cond(off > 128, compact,
                                lambda th2, of2: (th2, of2), theta, off)

            theta, off = lax.fori_loop(0, nch // 2, chunk2,
                                       (_NEG, np.int32(0)))
            top = top64_of_list(off)
            return jnp.maximum(lane(top[3], 2), theta)

        def phase_a(b):
            def step_a(g, carry):
                la, lb = carry

                def quad(c0):
                    va = jnp.maximum(rowbuf[b, pl.ds(c0 * 16, 16)],
                                     rowbuf[b, pl.ds((c0 + 1) * 16, 16)])
                    vb = jnp.maximum(rowbuf[b, pl.ds((c0 + 2) * 16, 16)],
                                     rowbuf[b, pl.ds((c0 + 3) * 16, 16)])
                    return jnp.maximum(va, vb)

                return (tuple(ins4(list(la), quad(8 * g))),
                        tuple(ins4(list(lb), quad(8 * g + 4))))

            la, lb = lax.fori_loop(0, nch // 8, step_a,
                                   ((neg16,) * 4, (neg16,) * 4), unroll=2)
            top4 = list(la)
            for r in lb:
                top4 = ins4(top4, r)
            return (bfly(top4[3], jnp.minimum),
                    bfly(top4[0], jnp.maximum))

        def collect(b, th):
            for j in range(BUF // 16):
                buf[pl.ds(j * 16, 16)] = neg16

            def step_b(i, addr):
                v = rowbuf[b, pl.ds(i * 16, 16)]
                msk = (v > th) & (addr < BUF)
                plsc.store_scatter(buf, [addr], v, mask=msk)
                return addr + jnp.where(msk, np.int32(16), np.int32(0))

            addr = lax.fori_loop(0, nch, step_b, iota, unroll=4)
            cnt = lax.shift_right_logical(addr - iota, 4)
            return bfly(cnt, jnp.add)[0], bfly(cnt, jnp.maximum)[0]

        def topsel(ov):
            def sel_from(nreg):
                def f():
                    regs = [buf[pl.ds(j * 16, 16)] for j in range(nreg)]
                    return tuple(_top64(regs))
                return f

            return lax.cond(ov <= 8, sel_from(8),
                            lambda: lax.cond(ov <= 16, sel_from(16),
                                             sel_from(STR)))

        def process(b, row_i, seed):
            seed_v = jnp.broadcast_to(seed, (16,))
            c_tot, ov = collect(b, seed_v)

            def fast():
                top = topsel(ov)
                t = lane(top[3], 2)
                r64 = lane(top[3], 15)
                nxt = jnp.maximum(6.0 * r64 - 5.0 * t, seed)
                return top[0], t, nxt

            def full():
                th, m1s = phase_a(b)
                c2, ov2 = collect(b, th)

                def t_main():
                    return lax.cond(
                        c2 > 50, lambda: lane(topsel(ov2)[3], 2),
                        lambda: th[0])

                t = lax.cond(ov2 >= STR, lambda: fallback_t(b), t_main)
                return m1s, t, th[0]

            m1v, t, nxt = lax.cond((c_tot > 50) & (ov < STR), fast, full)
            m1 = jnp.max(m1v)
            res = jnp.where(iota == 0, m1, jnp.where(iota == 1, t, 0.0))
            resv[row_i, :] = res
            return nxt

        def start(r, b):
            pltpu.async_copy(s_hbm.at[row0 + r], rowbuf.at[b], sems[b])

        def wait(b):
            pltpu.make_async_copy(s_hbm.at[0], rowbuf.at[b], sems[b]).wait()

        start(0, 0)

        def pair(g, seed):
            r = 2 * g
            start(r + 1, 1)
            wait(0)
            seed = process(0, r, seed)

            @pl.when(r + 2 < rpt)
            def _():
                start(r + 2, 0)

            wait(1)
            return process(1, r + 1, seed)

        lax.fori_loop(0, rpt // 2, pair, _NEG)
        pltpu.sync_copy(resv, res_hbm.at[pl.ds(row0, rpt)])

    return _sel(s_mat)


def _fd_body(s_ref, t_ref, m1_ref, x_ref, fd_ref, cs_ref):
    i, j = pl.program_id(0), pl.program_id(1)
    s = s_ref[...]
    t = t_ref[...].reshape(-1, 1)
    m1 = m1_ref[...].reshape(-1, 1)
    msk = ((s >= t) & (s != m1)).astype(jnp.float32)
    part = lax.dot_general(msk, x_ref[...], (((1,), (0,)), ((), ())),
                           preferred_element_type=jnp.float32) * (1.0 / K)

    @pl.when(j == 0)
    def _():
        fd_ref[...] = jnp.zeros_like(fd_ref)

    fd_ref[...] += part

    @pl.when((i == 0) & (j == 0))
    def _():
        cs_ref[...] = jnp.zeros_like(cs_ref)

    cs_ref[...] += jnp.sum(part, axis=0, keepdims=True)


def _neighbor_mean(s_mat, t, m1, x):
    n = x.shape[0]
    blk = 512
    return pl.pallas_call(
        _fd_body,
        grid=(n // blk, n // blk),
        in_specs=[
            pl.BlockSpec((blk, blk), lambda i, j: (i, j)),
            pl.BlockSpec((blk,), lambda i, j: (i,)),
            pl.BlockSpec((blk,), lambda i, j: (i,)),
            pl.BlockSpec((blk, D), lambda i, j: (j, 0)),
        ],
        out_specs=[
            pl.BlockSpec((blk, D), lambda i, j: (i, 0)),
            pl.BlockSpec((1, D), lambda i, j: (0, 0)),
        ],
        out_shape=[jax.ShapeDtypeStruct((n, D), jnp.float32),
                   jax.ShapeDtypeStruct((1, D), jnp.float32)],
    )(s_mat, t, m1, x)


def _sc_gather(table, idx):
    b = idx.shape[0]
    rpt = b // _NW
    mesh = plsc.VectorSubcoreMesh(core_axis_name="c", subcore_axis_name="s")

    @functools.partial(
        pl.kernel, mesh=mesh,
        out_type=jax.ShapeDtypeStruct((b, D), jnp.float32),
        scratch_types=[pltpu.VMEM((rpt,), jnp.int32),
                       pltpu.VMEM((rpt, D), jnp.float32),
                       pltpu.SemaphoreType.DMA])
    def _g(table_hbm, idx_hbm, out_hbm, idx_v, rows_v, sem):
        wid = lax.axis_index("s") * _NC + lax.axis_index("c")
        base = wid * rpt
        pltpu.sync_copy(idx_hbm.at[pl.ds(base, rpt)], idx_v)
        pltpu.async_copy(table_hbm.at[idx_v], rows_v, sem).wait()
        pltpu.sync_copy(rows_v, out_hbm.at[pl.ds(base, rpt)])

    return _g(table, idx)


def _row_cos(a, b):
    num = jnp.sum(a * b, axis=1)
    na = jnp.sqrt(jnp.sum(a * a, axis=1))
    nb = jnp.sqrt(jnp.sum(b * b, axis=1))
    return num / jnp.maximum(na * nb, 1e-8)


def _pred_body(hd_ref, hm_ref, w_ref, src_ref, dst_ref, pred_ref, ls_ref):
    hd = hd_ref[...]
    hm = hm_ref[...]
    p = lax.dot_general(hm, w_ref[...], (((1,), (0,)), ((), ())),
                        preferred_element_type=jnp.float32)
    pred_ref[...] = jax.nn.sigmoid(jnp.sum(hd * p, axis=1))
    pos = _row_cos(hd, src_ref[...])
    neg = _row_cos(hd, dst_ref[...])
    s = jnp.sum(jnp.log(jnp.exp(pos) + jnp.exp(neg)) - pos)

    @pl.when(pl.program_id(0) == 0)
    def _():
        ls_ref[...] = jnp.zeros_like(ls_ref)

    ls_ref[...] += s


def _pred_and_loss(h_d, h_m, w, src, dst):
    b = h_d.shape[0]
    blk = 512
    return pl.pallas_call(
        _pred_body,
        grid=(b // blk,),
        in_specs=[
            pl.BlockSpec((blk, D), lambda i: (i, 0)),
            pl.BlockSpec((blk, D), lambda i: (i, 0)),
            pl.BlockSpec((D, D), lambda i: (0, 0)),
            pl.BlockSpec((blk, D), lambda i: (i, 0)),
            pl.BlockSpec((blk, D), lambda i: (i, 0)),
        ],
        out_specs=[
            pl.BlockSpec((blk,), lambda i: (i,)),
            pl.BlockSpec((1, 128), lambda i: (0, 0)),
        ],
        out_shape=[jax.ShapeDtypeStruct((b,), jnp.float32),
                   jax.ShapeDtypeStruct((1, 128), jnp.float32)],
    )(h_d, h_m, w, src, dst)


def _floss_body(pos_div, neg_div, f_ref, csp_ref, csn_ref, ls_ref):
    f = f_ref[...]
    cp = csp_ref[...] * (1.0 / pos_div)
    cn = csn_ref[...] * (1.0 / neg_div)
    num_p = jnp.sum(f * cp, axis=1)
    num_n = jnp.sum(f * cn, axis=1)
    nf = jnp.sqrt(jnp.sum(f * f, axis=1))
    ncp = jnp.sqrt(jnp.sum(cp * cp))
    ncn = jnp.sqrt(jnp.sum(cn * cn))
    sp = num_p / jnp.maximum(nf * ncp, 1e-8)
    sn = num_n / jnp.maximum(nf * ncn, 1e-8)
    s = jnp.sum(jnp.log(jnp.exp(sp) + jnp.exp(sn)) - sp)

    @pl.when(pl.program_id(0) == 0)
    def _():
        ls_ref[...] = jnp.zeros_like(ls_ref)

    ls_ref[...] += s


def _feature_loss(f, cs_pos, pos_div, cs_neg, neg_div):
    n = f.shape[0]
    blk = 512
    return pl.pallas_call(
        functools.partial(_floss_body, pos_div, neg_div),
        grid=(n // blk,),
        in_specs=[
            pl.BlockSpec((blk, D), lambda i: (i, 0)),
            pl.BlockSpec((1, D), lambda i: (0, 0)),
            pl.BlockSpec((1, D), lambda i: (0, 0)),
        ],
        out_specs=pl.BlockSpec((1, 128), lambda i: (0, 0)),
        out_shape=jax.ShapeDtypeStruct((1, 128), jnp.float32),
    )(f, cs_pos, cs_neg)


def kernel(emb, h, src_init, dst_init, W, diseases, mirnas):
    xd = emb[:N_DISEASES]
    xm = emb[N_DISEASES:]

    s1 = _similarity(xd)
    s2 = _similarity(xm)
    res1 = _sc_select(s1)
    res2 = _sc_select(s2)
    m1_d, t_d = res1[:, 0], res1[:, 1]
    m1_m, t_m = res2[:, 0], res2[:, 1]
    fd, cs_d = _neighbor_mean(s1, t_d, m1_d, xd)
    fm, cs_m = _neighbor_mean(s2, t_m, m1_m, xm)

    h_d = _sc_gather(h, diseases.astype(jnp.int32))
    h_m = _sc_gather(h, mirnas.astype(jnp.int32))

    pred, closs_parts = _pred_and_loss(h_d, h_m, W, src_init, dst_init)
    contrastive_loss = closs_parts[0, 0] / h_d.shape[0]

    nd = float(fd.shape[0])
    nm = float(fm.shape[0])
    ld_parts = _feature_loss(fd, cs_d, nd, cs_m, nm)
    lm_parts = _feature_loss(fm, cs_m, nm, cs_d, nd)
    feature_contrastive_loss = (ld_parts[0, 0] / nd +
                                lm_parts[0, 0] / nm) / 2.0

    return (pred, contrastive_loss, feature_contrastive_loss)
